# baseline (speedup 1.0000x reference)
import functools

import jax
import jax.numpy as jnp
from jax import lax
from jax.experimental import pallas as pl
from jax.experimental.pallas import tpu as pltpu
from jax.experimental.pallas import tpu_sc as plsc

N = 10000
E = 320000
C = 128
H = 4

NTILES = 16
EPT = E // NTILES
GROUPS = EPT // 16

_SC_PARAMS = pltpu.CompilerParams(needs_layout_passes=False)
_MESH = plsc.VectorSubcoreMesh(core_axis_name="c", subcore_axis_name="s",
                               num_cores=1)


def _wid():
    return lax.axis_index("s")


@functools.partial(
    pl.kernel,
    out_type=jax.ShapeDtypeStruct((NTILES, N), jnp.float32),
    mesh=_MESH,
    compiler_params=_SC_PARAMS,
    scratch_types=[pltpu.VMEM((EPT,), jnp.int32),
                   pltpu.VMEM((N,), jnp.float32)],
)
def _sc_degree(dst_hbm, zeros_hbm, out_hbm, dst_v, acc_v):
    w = _wid()
    pltpu.sync_copy(zeros_hbm, acc_v)
    pltpu.sync_copy(dst_hbm.at[pl.ds(w * EPT, EPT)], dst_v)
    ones = jnp.ones((16,), jnp.float32)

    def body(i, carry):
        d = dst_v[pl.ds(i * 16, 16)]
        plsc.addupdate_scatter(acc_v, [d], ones)
        return carry

    lax.fori_loop(0, GROUPS, body, 0)
    pltpu.sync_copy(acc_v, out_hbm.at[w])


def _make_sc_agg(F):
    FN = F * N

    @functools.partial(
        pl.kernel,
        out_type=jax.ShapeDtypeStruct((NTILES, FN), jnp.float32),
        mesh=_MESH,
        compiler_params=_SC_PARAMS,
        scratch_types=[pltpu.VMEM((EPT,), jnp.int32),
                       pltpu.VMEM((EPT,), jnp.int32),
                       pltpu.VMEM((FN,), jnp.float32),
                       pltpu.VMEM((FN,), jnp.float32)],
    )
    def agg(q_hbm, src_hbm, dst_hbm, zeros_hbm, out_hbm,
            src_v, dst_v, q_v, acc_v):
        w = _wid()
        pltpu.sync_copy(q_hbm, q_v)
        pltpu.sync_copy(zeros_hbm, acc_v)
        pltpu.sync_copy(src_hbm.at[pl.ds(w * EPT, EPT)], src_v)
        pltpu.sync_copy(dst_hbm.at[pl.ds(w * EPT, EPT)], dst_v)

        def body(i, carry):
            s = src_v[pl.ds(i * 16, 16)]
            d = dst_v[pl.ds(i * 16, 16)]
            for j in range(F):
                si = s if j == 0 else s + (j * N)
                di = d if j == 0 else d + (j * N)
                g = plsc.load_gather(q_v, [si])
                plsc.addupdate_scatter(acc_v, [di], g)
            return carry

        lax.fori_loop(0, GROUPS, body, 0)
        pltpu.sync_copy(acc_v, out_hbm.at[w])

    return agg


_sc_agg4 = _make_sc_agg(H)
_sc_agg1 = _make_sc_agg(1)


def _tc1_body(degp_ref, x_ref, w1_ref, q1_ref, dinv_ref):
    deg = jnp.sum(degp_ref[...], axis=0, keepdims=True) + 1.0
    dinv = lax.rsqrt(deg)
    xwt = lax.dot_general(w1_ref[...], x_ref[...],
                          (((1,), (1,)), ((), ())),
                          preferred_element_type=jnp.float32)
    q1_ref[...] = xwt * dinv
    dinv_ref[...] = dinv


_tc1 = pl.pallas_call(
    _tc1_body,
    out_shape=(jax.ShapeDtypeStruct((H, N), jnp.float32),
               jax.ShapeDtypeStruct((1, N), jnp.float32)))


def _tc2_body(accp_ref, q1_ref, dinv_ref, b1_ref, w2_ref, q2_ref):
    acc = jnp.sum(accp_ref[...], axis=0) + q1_ref[...]
    dinv = dinv_ref[...]
    h = jnp.maximum(acc * dinv + b1_ref[...], 0.0)
    hwt = lax.dot_general(w2_ref[...], h, (((1,), (0,)), ((), ())),
                          preferred_element_type=jnp.float32)
    q2_ref[...] = hwt * dinv


_tc2 = pl.pallas_call(
    _tc2_body,
    out_shape=jax.ShapeDtypeStruct((1, N), jnp.float32))


def _tc3_body(accp_ref, q2_ref, dinv_ref, b2_ref, out_ref):
    acc = jnp.sum(accp_ref[...], axis=0, keepdims=True) + q2_ref[...]
    z = acc * dinv_ref[...] + b2_ref[...]
    out_ref[...] = 1.0 / (1.0 + jnp.exp(-z))


_tc3 = pl.pallas_call(
    _tc3_body,
    out_shape=jax.ShapeDtypeStruct((1, N), jnp.float32))


def kernel(x, edge_index, W1, b1, W2, b2):
    src = edge_index[0].astype(jnp.int32)
    dst = edge_index[1].astype(jnp.int32)
    zn = jnp.zeros((N,), jnp.float32)
    zhn = jnp.zeros((H * N,), jnp.float32)

    degp = _sc_degree(dst, zn)
    q1, dinv = _tc1(degp, x, W1)
    accp1 = _sc_agg4(q1.reshape(H * N), src, dst, zhn)
    q2 = _tc2(accp1.reshape(NTILES, H, N), q1, dinv,
              b1.reshape(H, 1), W2)
    accp2 = _sc_agg1(q2.reshape(N), src, dst, zn)
    out = _tc3(accp2, q2, dinv, b2.reshape(1, 1))
    return out.reshape(N, 1)


# speedup vs baseline: 1.3861x; 1.3861x over previous
import functools

import jax
import jax.numpy as jnp
from jax import lax
from jax.experimental import pallas as pl
from jax.experimental.pallas import tpu as pltpu
from jax.experimental.pallas import tpu_sc as plsc

N = 10000
E = 320000
C = 128
H = 4

NTILES = 32
EPT = E // NTILES
GROUPS = EPT // 16
UNROLL = 5

_SC_PARAMS = pltpu.CompilerParams(needs_layout_passes=False)
_MESH = plsc.VectorSubcoreMesh(core_axis_name="c", subcore_axis_name="s")


def _wid():
    return lax.axis_index("s") * 2 + lax.axis_index("c")


@functools.partial(
    pl.kernel,
    out_type=jax.ShapeDtypeStruct((NTILES, N), jnp.float32),
    mesh=_MESH,
    compiler_params=_SC_PARAMS,
    scratch_types=[pltpu.VMEM((EPT,), jnp.int32),
                   pltpu.VMEM((N,), jnp.float32),
                   pltpu.SemaphoreType.DMA,
                   pltpu.SemaphoreType.DMA],
)
def _sc_degree(dst_hbm, zeros_hbm, out_hbm, dst_v, acc_v, sem0, sem1):
    w = _wid()
    cp0 = pltpu.async_copy(zeros_hbm, acc_v, sem0)
    cp1 = pltpu.async_copy(dst_hbm.at[pl.ds(w * EPT, EPT)], dst_v, sem1)
    cp0.wait()
    cp1.wait()
    ones = jnp.ones((16,), jnp.float32)

    @plsc.parallel_loop(0, GROUPS, 1, unroll=UNROLL)
    def _(i):
        d = dst_v[pl.ds(i * 16, 16)]
        plsc.addupdate_scatter(acc_v, [d], ones)

    pltpu.sync_copy(acc_v, out_hbm.at[w])


def _make_sc_agg(F):
    FN = F * N

    @functools.partial(
        pl.kernel,
        out_type=jax.ShapeDtypeStruct((NTILES, FN), jnp.float32),
        mesh=_MESH,
        compiler_params=_SC_PARAMS,
        scratch_types=[pltpu.VMEM((EPT,), jnp.int32),
                       pltpu.VMEM((EPT,), jnp.int32),
                       pltpu.VMEM((FN,), jnp.float32),
                       pltpu.VMEM((FN,), jnp.float32),
                       pltpu.SemaphoreType.DMA,
                       pltpu.SemaphoreType.DMA,
                       pltpu.SemaphoreType.DMA,
                       pltpu.SemaphoreType.DMA],
    )
    def agg(q_hbm, src_hbm, dst_hbm, zeros_hbm, out_hbm,
            src_v, dst_v, q_v, acc_v, sem0, sem1, sem2, sem3):
        w = _wid()
        cp0 = pltpu.async_copy(q_hbm, q_v, sem0)
        cp1 = pltpu.async_copy(zeros_hbm, acc_v, sem1)
        cp2 = pltpu.async_copy(src_hbm.at[pl.ds(w * EPT, EPT)], src_v, sem2)
        cp3 = pltpu.async_copy(dst_hbm.at[pl.ds(w * EPT, EPT)], dst_v, sem3)
        cp0.wait()
        cp1.wait()
        cp2.wait()
        cp3.wait()

        @plsc.parallel_loop(0, GROUPS, 1, unroll=UNROLL)
        def _(i):
            s = src_v[pl.ds(i * 16, 16)]
            d = dst_v[pl.ds(i * 16, 16)]
            for j in range(F):
                si = s if j == 0 else s + (j * N)
                di = d if j == 0 else d + (j * N)
                g = plsc.load_gather(q_v, [si])
                plsc.addupdate_scatter(acc_v, [di], g)

        pltpu.sync_copy(acc_v, out_hbm.at[w])

    return agg


_sc_agg4 = _make_sc_agg(H)
_sc_agg1 = _make_sc_agg(1)


def _tc1_body(degp_ref, x_ref, w1_ref, q1_ref, dinv_ref):
    deg = jnp.sum(degp_ref[...], axis=0, keepdims=True) + 1.0
    dinv = lax.rsqrt(deg)
    xwt = lax.dot_general(w1_ref[...], x_ref[...],
                          (((1,), (1,)), ((), ())),
                          preferred_element_type=jnp.float32)
    q1_ref[...] = xwt * dinv
    dinv_ref[...] = dinv


_tc1 = pl.pallas_call(
    _tc1_body,
    out_shape=(jax.ShapeDtypeStruct((H, N), jnp.float32),
               jax.ShapeDtypeStruct((1, N), jnp.float32)))


def _tc2_body(accp_ref, q1_ref, dinv_ref, b1_ref, w2_ref, q2_ref):
    acc = jnp.sum(accp_ref[...], axis=0) + q1_ref[...]
    dinv = dinv_ref[...]
    h = jnp.maximum(acc * dinv + b1_ref[...], 0.0)
    hwt = lax.dot_general(w2_ref[...], h, (((1,), (0,)), ((), ())),
                          preferred_element_type=jnp.float32)
    q2_ref[...] = hwt * dinv


_tc2 = pl.pallas_call(
    _tc2_body,
    out_shape=jax.ShapeDtypeStruct((1, N), jnp.float32))


def _tc3_body(accp_ref, q2_ref, dinv_ref, b2_ref, out_ref):
    acc = jnp.sum(accp_ref[...], axis=0, keepdims=True) + q2_ref[...]
    z = acc * dinv_ref[...] + b2_ref[...]
    out_ref[...] = 1.0 / (1.0 + jnp.exp(-z))


_tc3 = pl.pallas_call(
    _tc3_body,
    out_shape=jax.ShapeDtypeStruct((1, N), jnp.float32))


def kernel(x, edge_index, W1, b1, W2, b2):
    src = edge_index[0].astype(jnp.int32)
    dst = edge_index[1].astype(jnp.int32)
    zn = jnp.zeros((N,), jnp.float32)
    zhn = jnp.zeros((H * N,), jnp.float32)

    degp = _sc_degree(dst, zn)
    q1, dinv = _tc1(degp, x, W1)
    accp1 = _sc_agg4(q1.reshape(H * N), src, dst, zhn)
    q2 = _tc2(accp1.reshape(NTILES, H, N), q1, dinv,
              b1.reshape(H, 1), W2)
    accp2 = _sc_agg1(q2.reshape(N), src, dst, zn)
    out = _tc3(accp2, q2, dinv, b2.reshape(1, 1))
    return out.reshape(N, 1)


# speedup vs baseline: 1.4001x; 1.0100x over previous
import functools

import jax
import jax.numpy as jnp
from jax import lax
from jax.experimental import pallas as pl
from jax.experimental.pallas import tpu as pltpu
from jax.experimental.pallas import tpu_sc as plsc

N = 10000
E = 320000
C = 128
H = 4

NTILES = 32
EPT = E // NTILES
GROUPS = EPT // 16
UNROLL = 5

_SC_PARAMS = pltpu.CompilerParams(needs_layout_passes=False)
_MESH = plsc.VectorSubcoreMesh(core_axis_name="c", subcore_axis_name="s")


def _wid():
    return lax.axis_index("s") * 2 + lax.axis_index("c")


@functools.partial(
    pl.kernel,
    out_type=jax.ShapeDtypeStruct((2, 1, N), jnp.float32),
    mesh=_MESH,
    compiler_params=_SC_PARAMS,
    scratch_types=[pltpu.VMEM((EPT,), jnp.int32),
                   pltpu.VMEM((1, N), jnp.float32),
                   pltpu.VMEM((1,), jnp.int32),
                   pltpu.VMEM_SHARED((1, N), jnp.float32),
                   pltpu.SemaphoreType.DMA,
                   pltpu.SemaphoreType.DMA,
                   pltpu.SemaphoreType.DMA],
)
def _sc_degree(dst_hbm, zeros2_hbm, zi_hbm, out_hbm,
               dst_v, acc_v, idx_v, shared, sem0, sem1, sem2):
    cc = lax.axis_index("c")
    ss = lax.axis_index("s")
    w = _wid()
    cp0 = pltpu.async_copy(zeros2_hbm, acc_v, sem0)
    cp1 = pltpu.async_copy(dst_hbm.at[pl.ds(w * EPT, EPT)], dst_v, sem1)
    cp2 = pltpu.async_copy(zi_hbm, idx_v, sem2)

    @pl.when(ss == 0)
    def _():
        pltpu.sync_copy(zeros2_hbm, shared)

    plsc.subcore_barrier()
    cp0.wait()
    cp1.wait()
    cp2.wait()
    acc = acc_v.at[0]
    ones = jnp.ones((16,), jnp.float32)

    @plsc.parallel_loop(0, GROUPS, 1, unroll=UNROLL)
    def _(i):
        d = dst_v[pl.ds(i * 16, 16)]
        plsc.addupdate_scatter(acc, [d], ones)

    pltpu.sync_copy(acc_v, shared.at[idx_v], add=True)
    plsc.subcore_barrier()

    @pl.when(ss == 0)
    def _():
        pltpu.sync_copy(shared, out_hbm.at[cc])


def _make_sc_agg(F):
    FN = F * N

    @functools.partial(
        pl.kernel,
        out_type=jax.ShapeDtypeStruct((2, 1, FN), jnp.float32),
        mesh=_MESH,
        compiler_params=_SC_PARAMS,
        scratch_types=[pltpu.VMEM((EPT,), jnp.int32),
                       pltpu.VMEM((EPT,), jnp.int32),
                       pltpu.VMEM((FN,), jnp.float32),
                       pltpu.VMEM((1, FN), jnp.float32),
                       pltpu.VMEM((1,), jnp.int32),
                       pltpu.VMEM_SHARED((1, FN), jnp.float32),
                       pltpu.SemaphoreType.DMA,
                       pltpu.SemaphoreType.DMA,
                       pltpu.SemaphoreType.DMA,
                       pltpu.SemaphoreType.DMA,
                       pltpu.SemaphoreType.DMA],
    )
    def agg(q_hbm, src_hbm, dst_hbm, zeros2_hbm, zi_hbm, out_hbm,
            src_v, dst_v, q_v, acc_v, idx_v, shared,
            sem0, sem1, sem2, sem3, sem4):
        cc = lax.axis_index("c")
        ss = lax.axis_index("s")
        w = _wid()
        cp0 = pltpu.async_copy(q_hbm, q_v, sem0)
        cp1 = pltpu.async_copy(zeros2_hbm, acc_v, sem1)
        cp2 = pltpu.async_copy(src_hbm.at[pl.ds(w * EPT, EPT)], src_v, sem2)
        cp3 = pltpu.async_copy(dst_hbm.at[pl.ds(w * EPT, EPT)], dst_v, sem3)
        cp4 = pltpu.async_copy(zi_hbm, idx_v, sem4)

        @pl.when(ss == 0)
        def _():
            pltpu.sync_copy(zeros2_hbm, shared)

        plsc.subcore_barrier()
        cp0.wait()
        cp1.wait()
        cp2.wait()
        cp3.wait()
        cp4.wait()
        acc = acc_v.at[0]

        @plsc.parallel_loop(0, GROUPS, 1, unroll=UNROLL)
        def _(i):
            s = src_v[pl.ds(i * 16, 16)]
            d = dst_v[pl.ds(i * 16, 16)]
            for j in range(F):
                si = s if j == 0 else s + (j * N)
                di = d if j == 0 else d + (j * N)
                g = plsc.load_gather(q_v, [si])
                plsc.addupdate_scatter(acc, [di], g)

        pltpu.sync_copy(acc_v, shared.at[idx_v], add=True)
        plsc.subcore_barrier()

        @pl.when(ss == 0)
        def _():
            pltpu.sync_copy(shared, out_hbm.at[cc])

    return agg


_sc_agg4 = _make_sc_agg(H)
_sc_agg1 = _make_sc_agg(1)


def _tc1_body(degp_ref, x_ref, w1_ref, q1_ref, dinv_ref):
    deg = jnp.sum(degp_ref[...], axis=0, keepdims=True) + 1.0
    dinv = lax.rsqrt(deg)
    xwt = lax.dot_general(w1_ref[...], x_ref[...],
                          (((1,), (1,)), ((), ())),
                          preferred_element_type=jnp.float32)
    q1_ref[...] = xwt * dinv
    dinv_ref[...] = dinv


_tc1 = pl.pallas_call(
    _tc1_body,
    out_shape=(jax.ShapeDtypeStruct((H, N), jnp.float32),
               jax.ShapeDtypeStruct((1, N), jnp.float32)))


def _tc2_body(accp_ref, q1_ref, dinv_ref, b1_ref, w2_ref, q2_ref):
    acc = jnp.sum(accp_ref[...], axis=0) + q1_ref[...]
    dinv = dinv_ref[...]
    h = jnp.maximum(acc * dinv + b1_ref[...], 0.0)
    hwt = lax.dot_general(w2_ref[...], h, (((1,), (0,)), ((), ())),
                          preferred_element_type=jnp.float32)
    q2_ref[...] = hwt * dinv


_tc2 = pl.pallas_call(
    _tc2_body,
    out_shape=jax.ShapeDtypeStruct((1, N), jnp.float32))


def _tc3_body(accp_ref, q2_ref, dinv_ref, b2_ref, out_ref):
    acc = jnp.sum(accp_ref[...], axis=0, keepdims=True) + q2_ref[...]
    z = acc * dinv_ref[...] + b2_ref[...]
    out_ref[...] = 1.0 / (1.0 + jnp.exp(-z))


_tc3 = pl.pallas_call(
    _tc3_body,
    out_shape=jax.ShapeDtypeStruct((1, N), jnp.float32))


def kernel(x, edge_index, W1, b1, W2, b2):
    src = edge_index[0].astype(jnp.int32)
    dst = edge_index[1].astype(jnp.int32)
    zn = jnp.zeros((1, N), jnp.float32)
    zhn = jnp.zeros((1, H * N), jnp.float32)
    zi = jnp.zeros((1,), jnp.int32)

    degp = _sc_degree(dst, zn, zi)
    q1, dinv = _tc1(degp.reshape(2, N), x, W1)
    accp1 = _sc_agg4(q1.reshape(H * N), src, dst, zhn, zi)
    q2 = _tc2(accp1.reshape(2, H, N), q1, dinv,
              b1.reshape(H, 1), W2)
    accp2 = _sc_agg1(q2.reshape(N), src, dst, zn, zi)
    out = _tc3(accp2.reshape(2, N), q2, dinv, b2.reshape(1, 1))
    return out.reshape(N, 1)


# speedup vs baseline: 1.4164x; 1.0117x over previous
import functools

import jax
import jax.numpy as jnp
from jax import lax
from jax.experimental import pallas as pl
from jax.experimental.pallas import tpu as pltpu
from jax.experimental.pallas import tpu_sc as plsc

N = 10000
E = 320000
C = 128
H = 4

NT = 16
NP = 10240
NS = NP // NT
SG = NS // 16
EPT = E // NT
CHUNK = 250
NCHUNK = EPT // (CHUNK * 16)

_SC_PARAMS = pltpu.CompilerParams(needs_layout_passes=False)
_MESH = plsc.VectorSubcoreMesh(core_axis_name="c", subcore_axis_name="s",
                               num_cores=1)


def _rsqrt_newton(x):
    i = plsc.bitcast(x, jnp.int32)
    i = jnp.int32(0x5F3759DF) - lax.shift_right_arithmetic(i, 1)
    y = plsc.bitcast(i, jnp.float32)
    for _ in range(3):
        y = y * (1.5 - 0.5 * x * y * y)
    return y


@functools.partial(
    pl.kernel,
    out_type=(jax.ShapeDtypeStruct((1, NP), jnp.float32),
              jax.ShapeDtypeStruct((1, H * NP), jnp.float32)),
    mesh=_MESH,
    compiler_params=_SC_PARAMS,
    scratch_types=[pltpu.VMEM((CHUNK * 16,), jnp.int32),
                   pltpu.VMEM((CHUNK * 16,), jnp.int32),
                   pltpu.VMEM((CHUNK * 16,), jnp.int32),
                   pltpu.VMEM((CHUNK * 16,), jnp.int32),
                   pltpu.VMEM((1, H * NP), jnp.float32),
                   pltpu.VMEM((1, H * NP), jnp.float32),
                   pltpu.VMEM((1, 6 * NS), jnp.float32),
                   pltpu.VMEM((1, H * NS), jnp.float32),
                   pltpu.VMEM((NS,), jnp.float32),
                   pltpu.VMEM((1, 144), jnp.float32),
                   pltpu.VMEM((1,), jnp.int32),
                   pltpu.VMEM_SHARED((1, NP), jnp.float32),
                   pltpu.VMEM_SHARED((1, H * NP), jnp.float32),
                   pltpu.SemaphoreType.DMA,
                   pltpu.SemaphoreType.DMA,
                   pltpu.SemaphoreType.DMA,
                   pltpu.SemaphoreType.DMA,
                   pltpu.SemaphoreType.DMA,
                   pltpu.SemaphoreType.DMA,
                   pltpu.SemaphoreType.DMA,
                   pltpu.SemaphoreType.DMA],
)
def _sc_gcn(xw_hbm, src_hbm, dst_hbm, z4_hbm, zn_hbm, params_hbm, zi_hbm,
            out_hbm, q1buf_hbm,
            src_v0, src_v1, dst_v0, dst_v1,
            q_v, acc_v, sl_v, q1s_v, dinv_v, par_v, idx_v,
            shn, sh4,
            semA, semD, semE, semF, semS0, semS1, semD0, semD1):
    t = lax.axis_index("s")
    n0 = t * NS
    ebase = t * EPT
    ones16 = jnp.ones((16,), jnp.float32)
    ssems = (semS0, semS1)
    dsems = (semD0, semD1)
    sbufs = (src_v0, src_v1)
    dbufs = (dst_v0, dst_v1)
    qf = q_v.at[0]
    accf = acc_v.at[0]
    slf = sl_v.at[0]
    q1sf = q1s_v.at[0]
    parf = par_v.at[0]

    def edge_stream(body, with_src):
        cps = [None, None]

        def fire(ci):
            b = ci % 2
            off = pl.ds(ebase + ci * CHUNK * 16, CHUNK * 16)
            cpd = pltpu.async_copy(dst_hbm.at[off], dbufs[b], dsems[b])
            cps_ = cpd
            if with_src:
                cps_ = (pltpu.async_copy(src_hbm.at[off], sbufs[b],
                                         ssems[b]), cpd)
            cps[b] = cps_

        fire(0)
        for ci in range(NCHUNK):
            if ci + 1 < NCHUNK:
                fire(ci + 1)
            got = cps[ci % 2]
            if with_src:
                got[0].wait()
                got[1].wait()
            else:
                got.wait()
            body(sbufs[ci % 2], dbufs[ci % 2])

    cpA = pltpu.async_copy(xw_hbm, q_v, semA)
    cpD = pltpu.async_copy(z4_hbm, acc_v, semD)
    cpE = pltpu.async_copy(params_hbm, par_v, semE)
    cpF = pltpu.async_copy(zi_hbm, idx_v, semF)

    @pl.when(t == 0)
    def _():
        pltpu.sync_copy(zn_hbm, shn)
        pltpu.sync_copy(z4_hbm, sh4)

    plsc.subcore_barrier()

    cpD.wait()
    cpF.wait()

    def deg_body(_sbuf, dbuf):
        @plsc.parallel_loop(0, CHUNK, 1, unroll=5)
        def _(i):
            d = dbuf[pl.ds(i * 16, 16)]
            plsc.addupdate_scatter(accf, [d], ones16)

    edge_stream(deg_body, with_src=False)

    pltpu.sync_copy(acc_v.at[:, pl.ds(0, NP)], shn.at[idx_v], add=True)
    cpD2 = pltpu.async_copy(z4_hbm, acc_v, semD)
    plsc.subcore_barrier()

    pltpu.sync_copy(shn.at[:, pl.ds(n0, NS)],
                    sl_v.at[:, pl.ds(5 * NS, NS)])
    cpA.wait()
    cpE.wait()
    for g in range(SG):
        o = g * 16
        deg = slf[pl.ds(5 * NS + o, 16)] + 1.0
        dv = _rsqrt_newton(deg)
        dinv_v[pl.ds(o, 16)] = dv
        for j in range(H):
            q1sf[pl.ds(j * NS + o, 16)] = dv * qf[pl.ds(j * NP + n0 + o, 16)]
    for j in range(H):
        pltpu.sync_copy(q1s_v.at[:, pl.ds(j * NS, NS)],
                        q1buf_hbm.at[:, pl.ds(j * NP + n0, NS)])
    plsc.subcore_barrier()

    pltpu.sync_copy(q1buf_hbm, q_v)
    cpD2.wait()

    def agg4_body(sbuf, dbuf):
        @plsc.parallel_loop(0, CHUNK, 1, unroll=4)
        def _(i):
            s = sbuf[pl.ds(i * 16, 16)]
            d = dbuf[pl.ds(i * 16, 16)]
            for j in range(H):
                si = s if j == 0 else s + (j * NP)
                di = d if j == 0 else d + (j * NP)
                g = plsc.load_gather(qf, [si])
                plsc.addupdate_scatter(accf, [di], g)

    edge_stream(agg4_body, with_src=True)

    pltpu.sync_copy(acc_v, sh4.at[idx_v], add=True)

    @pl.when(t == 0)
    def _():
        pltpu.sync_copy(zn_hbm, shn)

    plsc.subcore_barrier()

    for j in range(H):
        pltpu.sync_copy(sh4.at[:, pl.ds(j * NP + n0, NS)],
                        sl_v.at[:, pl.ds(j * NS, NS)])
    cpD3 = pltpu.async_copy(z4_hbm, acc_v, semD)
    for g in range(SG):
        o = g * 16
        dv = dinv_v[pl.ds(o, 16)]
        hw = jnp.zeros((16,), jnp.float32)
        for j in range(H):
            aj = slf[pl.ds(j * NS + o, 16)] + q1sf[pl.ds(j * NS + o, 16)]
            hj = jnp.maximum(dv * aj + parf[pl.ds(j * 16, 16)], 0.0)
            hw = hw + hj * parf[pl.ds((4 + j) * 16, 16)]
        slf[pl.ds(4 * NS + o, 16)] = dv * hw
    pltpu.sync_copy(sl_v.at[:, pl.ds(4 * NS, NS)],
                    q1buf_hbm.at[:, pl.ds(n0, NS)])
    plsc.subcore_barrier()

    pltpu.sync_copy(q1buf_hbm.at[:, pl.ds(0, NP)], q_v.at[:, pl.ds(0, NP)])
    cpD3.wait()

    def agg1_body(sbuf, dbuf):
        @plsc.parallel_loop(0, CHUNK, 1, unroll=5)
        def _(i):
            s = sbuf[pl.ds(i * 16, 16)]
            d = dbuf[pl.ds(i * 16, 16)]
            g = plsc.load_gather(qf, [s])
            plsc.addupdate_scatter(accf, [d], g)

    edge_stream(agg1_body, with_src=True)

    pltpu.sync_copy(acc_v.at[:, pl.ds(0, NP)], shn.at[idx_v], add=True)
    plsc.subcore_barrier()

    pltpu.sync_copy(shn.at[:, pl.ds(n0, NS)],
                    sl_v.at[:, pl.ds(5 * NS, NS)])
    for g in range(SG):
        o = g * 16
        dv = dinv_v[pl.ds(o, 16)]
        z = (dv * (slf[pl.ds(5 * NS + o, 16)] + slf[pl.ds(4 * NS + o, 16)])
             + parf[pl.ds(8 * 16, 16)])
        slf[pl.ds(3 * NS + o, 16)] = 1.0 / (1.0 + jnp.exp(-z))
    pltpu.sync_copy(sl_v.at[:, pl.ds(3 * NS, NS)],
                    out_hbm.at[:, pl.ds(n0, NS)])


def _tc0_body(x_ref, w1_ref, xwt_ref):
    xwt_ref[...] = lax.dot_general(w1_ref[...], x_ref[...],
                                   (((1,), (1,)), ((), ())),
                                   preferred_element_type=jnp.float32)


_tc0 = pl.pallas_call(
    _tc0_body,
    out_shape=jax.ShapeDtypeStruct((H, NP), jnp.float32))


def kernel(x, edge_index, W1, b1, W2, b2):
    src = edge_index[0].astype(jnp.int32)
    dst = edge_index[1].astype(jnp.int32)
    x_pad = jnp.pad(x, ((0, NP - N), (0, 0)))
    xwt = _tc0(x_pad, W1)

    z4 = jnp.zeros((1, H * NP), jnp.float32)
    zn = jnp.zeros((1, NP), jnp.float32)
    zi = jnp.zeros((1,), jnp.int32)
    params = jnp.concatenate(
        [jnp.broadcast_to(b1.reshape(H, 1), (H, 16)),
         jnp.broadcast_to(W2.reshape(H, 1), (H, 16)),
         jnp.broadcast_to(b2.reshape(1, 1), (1, 16))],
        axis=0).reshape(1, 144)

    out_pad, _ = _sc_gcn(xwt.reshape(1, H * NP), src, dst,
                         z4, zn, params, zi)
    return out_pad[0, :N].reshape(N, 1)


# speedup vs baseline: 1.4697x; 1.0376x over previous
import functools

import jax
import jax.numpy as jnp
from jax import lax
from jax.experimental import pallas as pl
from jax.experimental.pallas import tpu as pltpu
from jax.experimental.pallas import tpu_sc as plsc

N = 10000
E = 320000
C = 128
H = 4

NT = 16
NP = 10240
NS = NP // NT
SG = NS // 16
EPT = E // NT
CHUNK = 250
NCHUNK = EPT // (CHUNK * 16)

_SC_PARAMS = pltpu.CompilerParams(needs_layout_passes=False)
_MESH = plsc.VectorSubcoreMesh(core_axis_name="c", subcore_axis_name="s",
                               num_cores=1)


def _rsqrt_newton(x):
    i = plsc.bitcast(x, jnp.int32)
    i = jnp.int32(0x5F3759DF) - lax.shift_right_arithmetic(i, 1)
    y = plsc.bitcast(i, jnp.float32)
    for _ in range(3):
        y = y * (1.5 - 0.5 * x * y * y)
    return y


@functools.partial(
    pl.kernel,
    out_type=(jax.ShapeDtypeStruct((1, NP), jnp.float32),
              jax.ShapeDtypeStruct((1, H * NP), jnp.float32)),
    mesh=_MESH,
    compiler_params=_SC_PARAMS,
    scratch_types=[pltpu.VMEM((CHUNK * 16,), jnp.int32),
                   pltpu.VMEM((CHUNK * 16,), jnp.int32),
                   pltpu.VMEM((CHUNK * 16,), jnp.int32),
                   pltpu.VMEM((CHUNK * 16,), jnp.int32),
                   pltpu.VMEM((1, H * NP), jnp.float32),
                   pltpu.VMEM((1, H * NP), jnp.float32),
                   pltpu.VMEM((1, 6 * NS), jnp.float32),
                   pltpu.VMEM((1, H * NS), jnp.float32),
                   pltpu.VMEM((NS,), jnp.float32),
                   pltpu.VMEM((1, 144), jnp.float32),
                   pltpu.VMEM((1,), jnp.int32),
                   pltpu.VMEM_SHARED((1, NP), jnp.float32),
                   pltpu.VMEM_SHARED((1, H * NP), jnp.float32),
                   pltpu.SemaphoreType.DMA,
                   pltpu.SemaphoreType.DMA,
                   pltpu.SemaphoreType.DMA,
                   pltpu.SemaphoreType.DMA,
                   pltpu.SemaphoreType.DMA,
                   pltpu.SemaphoreType.DMA,
                   pltpu.SemaphoreType.DMA,
                   pltpu.SemaphoreType.DMA],
)
def _sc_gcn(xw_hbm, src_hbm, dst_hbm, z4_hbm, zn_hbm, params_hbm, zi_hbm,
            out_hbm, q1buf_hbm,
            src_v0, src_v1, dst_v0, dst_v1,
            q_v, acc_v, sl_v, q1s_v, dinv_v, par_v, idx_v,
            shn, sh4,
            semA, semD, semE, semF, semS0, semS1, semD0, semD1):
    t = lax.axis_index("s")
    n0 = t * NS
    ebase = t * EPT
    ones16 = jnp.ones((16,), jnp.float32)
    ssems = (semS0, semS1)
    dsems = (semD0, semD1)
    sbufs = (src_v0, src_v1)
    dbufs = (dst_v0, dst_v1)
    qf = q_v.at[0]
    accf = acc_v.at[0]
    slf = sl_v.at[0]
    q1sf = q1s_v.at[0]
    parf = par_v.at[0]

    def edge_stream(body, with_src):
        cps = [None, None]

        def fire(ci):
            b = ci % 2
            off = pl.ds(ebase + ci * CHUNK * 16, CHUNK * 16)
            cpd = pltpu.async_copy(dst_hbm.at[off], dbufs[b], dsems[b])
            cps_ = cpd
            if with_src:
                cps_ = (pltpu.async_copy(src_hbm.at[off], sbufs[b],
                                         ssems[b]), cpd)
            cps[b] = cps_

        fire(0)
        for ci in range(NCHUNK):
            if ci + 1 < NCHUNK:
                fire(ci + 1)
            got = cps[ci % 2]
            if with_src:
                got[0].wait()
                got[1].wait()
            else:
                got.wait()
            body(sbufs[ci % 2], dbufs[ci % 2])

    cpA = pltpu.async_copy(xw_hbm, q_v, semA)
    cpD = pltpu.async_copy(z4_hbm, acc_v, semD)
    cpE = pltpu.async_copy(params_hbm, par_v, semE)
    cpF = pltpu.async_copy(zi_hbm, idx_v, semF)

    @pl.when(t == 0)
    def _():
        pltpu.sync_copy(zn_hbm, shn)
        pltpu.sync_copy(z4_hbm, sh4)

    plsc.subcore_barrier()

    cpD.wait()
    cpF.wait()

    def deg_body(_sbuf, dbuf):
        @plsc.parallel_loop(0, CHUNK, 1, unroll=5)
        def _(i):
            d = dbuf[pl.ds(i * 16, 16)]
            plsc.addupdate_scatter(accf, [d], ones16)

    with jax.named_scope("phA_deg"):
        edge_stream(deg_body, with_src=False)

    with jax.named_scope("phA_red"):
        pltpu.sync_copy(acc_v.at[:, pl.ds(0, NP)], shn.at[idx_v],
                        add=True)
    cpD2 = pltpu.async_copy(z4_hbm, acc_v, semD)
    plsc.subcore_barrier()

    pltpu.sync_copy(shn.at[:, pl.ds(n0, NS)],
                    sl_v.at[:, pl.ds(5 * NS, NS)])
    cpA.wait()
    cpE.wait()
    for g in range(SG):
        o = g * 16
        deg = slf[pl.ds(5 * NS + o, 16)] + 1.0
        dv = _rsqrt_newton(deg)
        dinv_v[pl.ds(o, 16)] = dv
        for j in range(H):
            q1sf[pl.ds(j * NS + o, 16)] = dv * qf[pl.ds(j * NP + n0 + o, 16)]
    for j in range(H):
        pltpu.sync_copy(q1s_v.at[:, pl.ds(j * NS, NS)],
                        q1buf_hbm.at[:, pl.ds(j * NP + n0, NS)])
    plsc.subcore_barrier()

    with jax.named_scope("phB_q1rd"):
        pltpu.sync_copy(q1buf_hbm, q_v)
    cpD2.wait()

    def agg4_body(sbuf, dbuf):
        @plsc.parallel_loop(0, CHUNK, 1, unroll=4)
        def _(i):
            s = sbuf[pl.ds(i * 16, 16)]
            d = dbuf[pl.ds(i * 16, 16)]
            for j in range(H):
                si = s if j == 0 else s + (j * NP)
                di = d if j == 0 else d + (j * NP)
                g = plsc.load_gather(qf, [si])
                plsc.addupdate_scatter(accf, [di], g)

    with jax.named_scope("phB_edges"):
        edge_stream(agg4_body, with_src=True)

    with jax.named_scope("phB_red"):
        pltpu.sync_copy(acc_v, sh4.at[idx_v], add=True)

    @pl.when(t == 0)
    def _():
        pltpu.sync_copy(zn_hbm, shn)

    plsc.subcore_barrier()

    for j in range(H):
        pltpu.sync_copy(sh4.at[:, pl.ds(j * NP + n0, NS)],
                        sl_v.at[:, pl.ds(j * NS, NS)])
    cpD3 = pltpu.async_copy(z4_hbm, acc_v, semD)
    for g in range(SG):
        o = g * 16
        dv = dinv_v[pl.ds(o, 16)]
        hw = jnp.zeros((16,), jnp.float32)
        for j in range(H):
            aj = slf[pl.ds(j * NS + o, 16)] + q1sf[pl.ds(j * NS + o, 16)]
            hj = jnp.maximum(dv * aj + parf[pl.ds(j * 16, 16)], 0.0)
            hw = hw + hj * parf[pl.ds((4 + j) * 16, 16)]
        slf[pl.ds(4 * NS + o, 16)] = dv * hw
    pltpu.sync_copy(sl_v.at[:, pl.ds(4 * NS, NS)],
                    q1buf_hbm.at[:, pl.ds(n0, NS)])
    plsc.subcore_barrier()

    pltpu.sync_copy(q1buf_hbm.at[:, pl.ds(0, NP)], q_v.at[:, pl.ds(0, NP)])
    cpD3.wait()

    def agg1_body(sbuf, dbuf):
        @plsc.parallel_loop(0, CHUNK, 1, unroll=5)
        def _(i):
            s = sbuf[pl.ds(i * 16, 16)]
            d = dbuf[pl.ds(i * 16, 16)]
            g = plsc.load_gather(qf, [s])
            plsc.addupdate_scatter(accf, [d], g)

    edge_stream(agg1_body, with_src=True)

    pltpu.sync_copy(acc_v.at[:, pl.ds(0, NP)], shn.at[idx_v], add=True)
    plsc.subcore_barrier()

    pltpu.sync_copy(shn.at[:, pl.ds(n0, NS)],
                    sl_v.at[:, pl.ds(5 * NS, NS)])
    for g in range(SG):
        o = g * 16
        dv = dinv_v[pl.ds(o, 16)]
        z = (dv * (slf[pl.ds(5 * NS + o, 16)] + slf[pl.ds(4 * NS + o, 16)])
             + parf[pl.ds(8 * 16, 16)])
        slf[pl.ds(3 * NS + o, 16)] = 1.0 / (1.0 + jnp.exp(-z))
    pltpu.sync_copy(sl_v.at[:, pl.ds(3 * NS, NS)],
                    out_hbm.at[:, pl.ds(n0, NS)])


def _tc0_body(x_ref, w1_ref, xwt_ref):
    xwt_ref[...] = jnp.zeros((H, NP), jnp.float32)
    xwt_ref[:, :N] = lax.dot_general(w1_ref[...], x_ref[...],
                                     (((1,), (1,)), ((), ())),
                                     preferred_element_type=jnp.float32)


_tc0 = pl.pallas_call(
    _tc0_body,
    out_shape=jax.ShapeDtypeStruct((H, NP), jnp.float32))


def kernel(x, edge_index, W1, b1, W2, b2):
    src = edge_index[0].astype(jnp.int32)
    dst = edge_index[1].astype(jnp.int32)
    xwt = _tc0(x, W1)

    z4 = jnp.zeros((1, H * NP), jnp.float32)
    zn = jnp.zeros((1, NP), jnp.float32)
    zi = jnp.zeros((1,), jnp.int32)
    params = jnp.concatenate(
        [jnp.broadcast_to(b1.reshape(H, 1), (H, 16)),
         jnp.broadcast_to(W2.reshape(H, 1), (H, 16)),
         jnp.broadcast_to(b2.reshape(1, 1), (1, 16))],
        axis=0).reshape(1, 144)

    out_pad, _ = _sc_gcn(xwt.reshape(1, H * NP), src, dst,
                         z4, zn, params, zi)
    return out_pad[0, :N].reshape(N, 1)


# speedup vs baseline: 1.7726x; 1.2061x over previous
import functools

import jax
import jax.numpy as jnp
from jax import lax
from jax.experimental import pallas as pl
from jax.experimental.pallas import tpu as pltpu
from jax.experimental.pallas import tpu_sc as plsc

N = 10000
E = 320000
C = 128
H = 4

NT = 16
NP = 10240
NS = NP // NT
SG = NS // 16
EPT = E // NT
CHUNK = 250
NCHUNK = EPT // (CHUNK * 16)

_SC_PARAMS = pltpu.CompilerParams(needs_layout_passes=False)
_MESH = plsc.VectorSubcoreMesh(core_axis_name="c", subcore_axis_name="s",
                               num_cores=1)


def _rsqrt_newton(x):
    i = plsc.bitcast(x, jnp.int32)
    i = jnp.int32(0x5F3759DF) - lax.shift_right_arithmetic(i, 1)
    y = plsc.bitcast(i, jnp.float32)
    for _ in range(3):
        y = y * (1.5 - 0.5 * x * y * y)
    return y


@functools.partial(
    pl.kernel,
    out_type=jax.ShapeDtypeStruct((1, NP), jnp.float32),
    mesh=_MESH,
    compiler_params=_SC_PARAMS,
    scratch_types=[pltpu.VMEM((CHUNK * 16,), jnp.int32),
                   pltpu.VMEM((CHUNK * 16,), jnp.int32),
                   pltpu.VMEM((CHUNK * 16,), jnp.int32),
                   pltpu.VMEM((CHUNK * 16,), jnp.int32),
                   pltpu.VMEM((1, H * NP), jnp.float32),
                   pltpu.VMEM((1, H * NP), jnp.float32),
                   pltpu.VMEM((1, 6 * NS), jnp.float32),
                   pltpu.VMEM((1, H * NS), jnp.float32),
                   pltpu.VMEM((NS,), jnp.float32),
                   pltpu.VMEM((1, 144), jnp.float32),
                   pltpu.VMEM((1,), jnp.int32),
                   pltpu.VMEM_SHARED((1, NP), jnp.float32),
                   pltpu.VMEM_SHARED((1, H * NP), jnp.float32),
                   pltpu.VMEM_SHARED((1, H * NP), jnp.float32),
                   pltpu.SemaphoreType.DMA,
                   pltpu.SemaphoreType.DMA,
                   pltpu.SemaphoreType.DMA,
                   pltpu.SemaphoreType.DMA,
                   pltpu.SemaphoreType.DMA,
                   pltpu.SemaphoreType.DMA,
                   pltpu.SemaphoreType.DMA,
                   pltpu.SemaphoreType.DMA],
)
def _sc_gcn(xw_hbm, ei_hbm, z4_hbm, zn_hbm, params_hbm, zi_hbm,
            out_hbm,
            src_v0, src_v1, dst_v0, dst_v1,
            q_v, acc_v, sl_v, q1s_v, dinv_v, par_v, idx_v,
            shn, sh4, shq,
            semA, semD, semE, semF, semS0, semS1, semD0, semD1):
    t = lax.axis_index("s")
    n0 = t * NS
    ebase = t * EPT
    ones16 = jnp.ones((16,), jnp.float32)
    ssems = (semS0, semS1)
    dsems = (semD0, semD1)
    sbufs = (src_v0, src_v1)
    dbufs = (dst_v0, dst_v1)
    qf = q_v.at[0]
    accf = acc_v.at[0]
    slf = sl_v.at[0]
    q1sf = q1s_v.at[0]
    parf = par_v.at[0]

    def edge_stream(body, with_src):
        cps = [None, None]

        def fire(ci):
            b = ci % 2
            o0 = ebase + ci * CHUNK * 16
            cpd = pltpu.async_copy(ei_hbm.at[pl.ds(E + o0, CHUNK * 16)],
                                   dbufs[b], dsems[b])
            cps_ = cpd
            if with_src:
                cps_ = (pltpu.async_copy(ei_hbm.at[pl.ds(o0, CHUNK * 16)],
                                         sbufs[b], ssems[b]), cpd)
            cps[b] = cps_

        fire(0)
        for ci in range(NCHUNK):
            if ci + 1 < NCHUNK:
                fire(ci + 1)
            got = cps[ci % 2]
            if with_src:
                got[0].wait()
                got[1].wait()
            else:
                got.wait()
            body(sbufs[ci % 2], dbufs[ci % 2])

    cpA = pltpu.async_copy(xw_hbm, q_v, semA)
    cpD = pltpu.async_copy(z4_hbm, acc_v, semD)
    cpE = pltpu.async_copy(params_hbm, par_v, semE)
    cpF = pltpu.async_copy(zi_hbm, idx_v, semF)

    @pl.when(t == 0)
    def _():
        pltpu.sync_copy(zn_hbm, shn)
        pltpu.sync_copy(z4_hbm, sh4)

    plsc.subcore_barrier()

    cpD.wait()
    cpF.wait()

    def deg_body(_sbuf, dbuf):
        @plsc.parallel_loop(0, CHUNK, 1, unroll=5)
        def _(i):
            d = dbuf[pl.ds(i * 16, 16)]
            plsc.addupdate_scatter(accf, [d], ones16)

    with jax.named_scope("phA_deg"):
        edge_stream(deg_body, with_src=False)

    with jax.named_scope("phA_red"):
        pltpu.sync_copy(acc_v.at[:, pl.ds(0, NP)], shn.at[idx_v],
                        add=True)
    cpD2 = pltpu.async_copy(z4_hbm, acc_v, semD)
    plsc.subcore_barrier()

    pltpu.sync_copy(shn.at[:, pl.ds(n0, NS)],
                    sl_v.at[:, pl.ds(5 * NS, NS)])
    cpA.wait()
    cpE.wait()
    def a2_body(g, c):
        o = g * 16
        deg = slf[pl.ds(5 * NS + o, 16)] + 1.0
        dv = _rsqrt_newton(deg)
        dinv_v[pl.ds(o, 16)] = dv
        for j in range(H):
            q1sf[pl.ds(j * NS + o, 16)] = dv * qf[pl.ds(j * NP + n0 + o, 16)]
        return c

    lax.fori_loop(0, SG, a2_body, 0)
    for j in range(H):
        pltpu.sync_copy(q1s_v.at[:, pl.ds(j * NS, NS)],
                        shq.at[:, pl.ds(j * NP + n0, NS)])
    plsc.subcore_barrier()

    with jax.named_scope("phB_q1rd"):
        pltpu.sync_copy(shq, q_v)
    cpD2.wait()

    def agg4_body(sbuf, dbuf):
        @plsc.parallel_loop(0, CHUNK, 1, unroll=4)
        def _(i):
            s = sbuf[pl.ds(i * 16, 16)]
            d = dbuf[pl.ds(i * 16, 16)]
            for j in range(H):
                si = s if j == 0 else s + (j * NP)
                di = d if j == 0 else d + (j * NP)
                g = plsc.load_gather(qf, [si])
                plsc.addupdate_scatter(accf, [di], g)

    with jax.named_scope("phB_edges"):
        edge_stream(agg4_body, with_src=True)

    with jax.named_scope("phB_red"):
        pltpu.sync_copy(acc_v, sh4.at[idx_v], add=True)

    @pl.when(t == 0)
    def _():
        pltpu.sync_copy(zn_hbm, shn)

    plsc.subcore_barrier()

    for j in range(H):
        pltpu.sync_copy(sh4.at[:, pl.ds(j * NP + n0, NS)],
                        sl_v.at[:, pl.ds(j * NS, NS)])
    cpD3 = pltpu.async_copy(z4_hbm, acc_v, semD)

    def b2_body(g, c):
        o = g * 16
        dv = dinv_v[pl.ds(o, 16)]
        hw = jnp.zeros((16,), jnp.float32)
        for j in range(H):
            aj = slf[pl.ds(j * NS + o, 16)] + q1sf[pl.ds(j * NS + o, 16)]
            hj = jnp.maximum(dv * aj + parf[pl.ds(j * 16, 16)], 0.0)
            hw = hw + hj * parf[pl.ds((4 + j) * 16, 16)]
        slf[pl.ds(4 * NS + o, 16)] = dv * hw
        return c

    lax.fori_loop(0, SG, b2_body, 0)
    pltpu.sync_copy(sl_v.at[:, pl.ds(4 * NS, NS)],
                    shq.at[:, pl.ds(n0, NS)])
    plsc.subcore_barrier()

    pltpu.sync_copy(shq.at[:, pl.ds(0, NP)], q_v.at[:, pl.ds(0, NP)])
    cpD3.wait()

    def agg1_body(sbuf, dbuf):
        @plsc.parallel_loop(0, CHUNK, 1, unroll=5)
        def _(i):
            s = sbuf[pl.ds(i * 16, 16)]
            d = dbuf[pl.ds(i * 16, 16)]
            g = plsc.load_gather(qf, [s])
            plsc.addupdate_scatter(accf, [d], g)

    edge_stream(agg1_body, with_src=True)

    pltpu.sync_copy(acc_v.at[:, pl.ds(0, NP)], shn.at[idx_v], add=True)
    plsc.subcore_barrier()

    pltpu.sync_copy(shn.at[:, pl.ds(n0, NS)],
                    sl_v.at[:, pl.ds(5 * NS, NS)])
    def c2_body(g, c):
        o = g * 16
        dv = dinv_v[pl.ds(o, 16)]
        z = (dv * (slf[pl.ds(5 * NS + o, 16)] + slf[pl.ds(4 * NS + o, 16)])
             + parf[pl.ds(8 * 16, 16)])
        slf[pl.ds(3 * NS + o, 16)] = 1.0 / (1.0 + jnp.exp(-z))
        return c

    lax.fori_loop(0, SG, c2_body, 0)
    pltpu.sync_copy(sl_v.at[:, pl.ds(3 * NS, NS)],
                    out_hbm.at[:, pl.ds(n0, NS)])


def _tc0_body(x_ref, w1_ref, xwt_ref):
    xwt_ref[...] = jnp.zeros((H, NP), jnp.float32)
    xwt_ref[:, :N] = lax.dot_general(w1_ref[...], x_ref[...],
                                     (((1,), (1,)), ((), ())),
                                     preferred_element_type=jnp.float32)


_tc0 = pl.pallas_call(
    _tc0_body,
    out_shape=jax.ShapeDtypeStruct((H, NP), jnp.float32))


def kernel(x, edge_index, W1, b1, W2, b2):
    ei = edge_index.astype(jnp.int32).reshape(2 * E)
    xwt = _tc0(x, W1)

    z4 = jnp.zeros((1, H * NP), jnp.float32)
    zn = jnp.zeros((1, NP), jnp.float32)
    zi = jnp.zeros((1,), jnp.int32)
    params = jnp.concatenate(
        [jnp.broadcast_to(b1.reshape(H, 1), (H, 16)),
         jnp.broadcast_to(W2.reshape(H, 1), (H, 16)),
         jnp.broadcast_to(b2.reshape(1, 1), (1, 16))],
        axis=0).reshape(1, 144)

    out_pad = _sc_gcn(xwt.reshape(1, H * NP), ei,
                      z4, zn, params, zi)
    return out_pad[0, :N].reshape(N, 1)
